# baseline (device time: 57029 ns/iter reference)
import jax
import jax.numpy as jnp
from jax import lax
from jax.experimental import pallas as pl
from jax.experimental.pallas import tpu as pltpu

N_DEV = 4
SCALE = 128 ** -0.5


def _fused_body(
    q_ref, k_ref, v_ref, out_ref,
    lacc, oacc, obuf, lbuf, o_comm, l_comm,
    kbuf, vbuf, ksem, vsem,
    o_ssem, o_rsem, l_ssem, l_rsem,
):
    bi = pl.program_id(0)
    ki = pl.program_id(1)
    nb = pl.num_programs(0)
    nk = pl.num_programs(1)
    my = lax.axis_index("i")
    n_heads = q_ref.shape[2]
    kc = kbuf.shape[2]
    t = bi * nk + ki
    nsteps = nb * nk

    @pl.when((bi == 0) & (ki == 0))
    def _():
        barrier = pltpu.get_barrier_semaphore()
        for delta in range(1, N_DEV):
            pl.semaphore_signal(
                barrier,
                inc=1,
                device_id=((my + delta) % N_DEV,),
                device_id_type=pl.DeviceIdType.MESH,
            )
        pl.semaphore_wait(barrier, N_DEV - 1)

    @pl.when(ki == 0)
    def _():
        lacc[...] = jnp.zeros_like(lacc)
        oacc[...] = jnp.zeros_like(oacc)

    def _issue(tt):
        bi2 = tt // nk
        ki2 = tt % nk
        slot = tt % 2
        for j in range(n_heads):
            pltpu.make_async_copy(
                k_ref.at[bi2, pl.ds(ki2 * kc, kc), j], kbuf.at[slot, j], ksem.at[slot, j]
            ).start()
            pltpu.make_async_copy(
                v_ref.at[bi2, pl.ds(ki2 * kc, kc), j], vbuf.at[slot, j], vsem.at[slot, j]
            ).start()

    @pl.when(t == 0)
    def _():
        _issue(t)

    @pl.when(t + 1 < nsteps)
    def _():
        _issue(t + 1)

    slot = t % 2
    q_all = q_ref[0] * SCALE

    for j in range(n_heads):
        pltpu.make_async_copy(
            k_ref.at[bi, pl.ds(ki * kc, kc), j], kbuf.at[slot, j], ksem.at[slot, j]
        ).wait()
        pltpu.make_async_copy(
            v_ref.at[bi, pl.ds(ki * kc, kc), j], vbuf.at[slot, j], vsem.at[slot, j]
        ).wait()
        q = q_all[:, j, :]
        k = kbuf[slot, j]
        v = vbuf[slot, j]
        s = lax.dot_general(
            q, k, (((1,), (1,)), ((), ())), preferred_element_type=jnp.float32
        )
        p = jnp.exp(s)
        pv = lax.dot_general(
            p, v, (((1,), (0,)), ((), ())), preferred_element_type=jnp.float32
        )
        lacc[:, j : j + 1] = lacc[:, j : j + 1] + jnp.sum(p, axis=-1, keepdims=True)
        oacc[:, j, :] = oacc[:, j, :] + pv

    @pl.when(ki == nk - 1)
    def _():
        obuf[bi] = oacc[...].astype(jnp.bfloat16)
        lbuf[bi] = lacc[...]
        for delta in range(1, N_DEV):
            tgt = (my + delta) % N_DEV
            slot = 3 - delta
            pltpu.make_async_remote_copy(
                src_ref=obuf.at[bi],
                dst_ref=o_comm.at[slot, bi],
                send_sem=o_ssem.at[delta - 1, bi],
                recv_sem=o_rsem.at[slot, bi],
                device_id=(tgt,),
                device_id_type=pl.DeviceIdType.MESH,
            ).start()
            pltpu.make_async_remote_copy(
                src_ref=lbuf.at[bi],
                dst_ref=l_comm.at[slot, bi],
                send_sem=l_ssem.at[delta - 1, bi],
                recv_sem=l_rsem.at[slot, bi],
                device_id=(tgt,),
                device_id_type=pl.DeviceIdType.MESH,
            ).start()

    @pl.when((bi == nb - 1) & (ki == nk - 1))
    def _():
        def _o_desc(slot, bb, send_idx=0):
            return pltpu.make_async_remote_copy(
                src_ref=obuf.at[bb],
                dst_ref=o_comm.at[slot, bb],
                send_sem=o_ssem.at[send_idx, bb],
                recv_sem=o_rsem.at[slot, bb],
                device_id=(my,),
                device_id_type=pl.DeviceIdType.MESH,
            )

        def _l_desc(slot, bb, send_idx=0):
            return pltpu.make_async_remote_copy(
                src_ref=lbuf.at[bb],
                dst_ref=l_comm.at[slot, bb],
                send_sem=l_ssem.at[send_idx, bb],
                recv_sem=l_rsem.at[slot, bb],
                device_id=(my,),
                device_id_type=pl.DeviceIdType.MESH,
            )

        for slot in range(N_DEV - 1):
            for bb in range(N_DEV):
                _o_desc(slot, bb).wait_recv()
                _l_desc(slot, bb).wait_recv()

        l_g = lbuf[...]
        o_g = obuf[...].astype(jnp.float32)
        for slot in range(N_DEV - 1):
            l_g = l_g + l_comm[slot]
            o_g = o_g + o_comm[slot].astype(jnp.float32)
        out_ref[...] = o_g / l_g[..., None]

        for delta in range(1, N_DEV):
            for bb in range(N_DEV):
                _o_desc(3 - delta, bb, send_idx=delta - 1).wait_send()
                _l_desc(3 - delta, bb, send_idx=delta - 1).wait_send()


def kernel(Q, K, V):
    b, sq, h, d = Q.shape
    kv = K.shape[1]
    kc = 1024
    nk = kv // kc

    return pl.pallas_call(
        _fused_body,
        grid=(b, nk),
        in_specs=[
            pl.BlockSpec((1, sq, h, d), lambda i, ki: (i, 0, 0, 0)),
            pl.BlockSpec(memory_space=pl.ANY),
            pl.BlockSpec(memory_space=pl.ANY),
        ],
        out_specs=pl.BlockSpec(memory_space=pltpu.VMEM),
        out_shape=jax.ShapeDtypeStruct((b, sq, h, d), jnp.float32),
        scratch_shapes=[
            pltpu.VMEM((sq, h), jnp.float32),
            pltpu.VMEM((sq, h, d), jnp.float32),
            pltpu.VMEM((b, sq, h, d), jnp.bfloat16),
            pltpu.VMEM((b, sq, h), jnp.float32),
            pltpu.VMEM((N_DEV - 1, b, sq, h, d), jnp.bfloat16),
            pltpu.VMEM((N_DEV - 1, b, sq, h), jnp.float32),
            pltpu.VMEM((2, h, kc, d), jnp.float32),
            pltpu.VMEM((2, h, kc, d), jnp.float32),
            pltpu.SemaphoreType.DMA((2, h)),
            pltpu.SemaphoreType.DMA((2, h)),
            pltpu.SemaphoreType.DMA((N_DEV - 1, 4)),
            pltpu.SemaphoreType.DMA((N_DEV - 1, 4)),
            pltpu.SemaphoreType.DMA((N_DEV - 1, 4)),
            pltpu.SemaphoreType.DMA((N_DEV - 1, 4)),
        ],
        compiler_params=pltpu.CompilerParams(collective_id=0),
    )(Q, K, V)
